# baseline (speedup 1.0000x reference)
import functools
import jax
import jax.numpy as jnp
from jax import lax
from jax.experimental import pallas as pl
from jax.experimental.pallas import tpu as pltpu
from jax.experimental.pallas import tpu_sc as plsc

DD = 128
SN = 40
DOC_BLK = 8
NC = 2
NS = 16
NW = NC * NS

_INTERPRET = False


def _wsrc_chain_body(x_ref, w0_ref, w1_ref, w2_ref, o0_ref, o1_ref, o2_ref):
    x = x_ref[...]
    a0 = jnp.dot(x, w0_ref[...], preferred_element_type=jnp.float32)
    o0_ref[...] = a0
    a1 = jnp.dot(jnp.maximum(a0, 0.0), w1_ref[...],
                 preferred_element_type=jnp.float32)
    o1_ref[...] = a1
    o2_ref[...] = jnp.dot(jnp.maximum(a1, 0.0), w2_ref[...],
                          preferred_element_type=jnp.float32)


def _wsrc_chain(word_h, Ww0, Ww1, Ww2):
    n = word_h.shape[0]
    blk = 1024
    grid = n // blk
    w_spec = pl.BlockSpec((DD, DD), lambda i: (0, 0))
    x_spec = pl.BlockSpec((blk, DD), lambda i: (i, 0))
    shp = jax.ShapeDtypeStruct((n, DD), jnp.float32)
    return pl.pallas_call(
        _wsrc_chain_body,
        grid=(grid,),
        in_specs=[x_spec, w_spec, w_spec, w_spec],
        out_specs=[x_spec, x_spec, x_spec],
        out_shape=[shp, shp, shp],
        interpret=_INTERPRET,
    )(word_h, Ww0, Ww1, Ww2)


def _init_super_body(sent_ref, score_ref, ws_ref, sup_ref, sdst_ref):
    sent = sent_ref[...]
    sc = score_ref[...]
    s3 = sent.reshape(DOC_BLK, SN, DD)
    sup_ref[...] = lax.dot_general(sc, s3, (((1,), (1,)), ((0,), (0,))))
    sdst_ref[...] = jnp.dot(sent, ws_ref[...],
                            preferred_element_type=jnp.float32)


def _init_super(sent_h, score, Ws0):
    n = sent_h.shape[0]
    Bn = score.shape[0]
    grid = Bn // DOC_BLK
    return pl.pallas_call(
        _init_super_body,
        grid=(grid,),
        in_specs=[
            pl.BlockSpec((DOC_BLK * SN, DD), lambda i: (i, 0)),
            pl.BlockSpec((DOC_BLK, SN), lambda i: (i, 0)),
            pl.BlockSpec((DD, DD), lambda i: (0, 0)),
        ],
        out_specs=[
            pl.BlockSpec((DOC_BLK, DD), lambda i: (i, 0)),
            pl.BlockSpec((DOC_BLK * SN, DD), lambda i: (i, 0)),
        ],
        out_shape=[
            jax.ShapeDtypeStruct((Bn, DD), jnp.float32),
            jax.ShapeDtypeStruct((n, DD), jnp.float32),
        ],
        interpret=_INTERPRET,
    )(sent_h, score, Ws0)


def _super_layer_body(sent_ref, sup_ref, wsup_ref, wsn_ref,
                      score_ref, supo_ref, sdst_ref):
    sent = sent_ref[...]
    sup = sup_ref[...]
    proj = jnp.dot(sup, wsup_ref[...], preferred_element_type=jnp.float32)
    s3 = sent.reshape(DOC_BLK, SN, DD)
    logits = lax.dot_general(s3, proj, (((2,), (1,)), ((0,), (0,))))
    logits = logits * (DD ** -0.5)
    logits = jnp.where(logits >= 0, logits, 0.01 * logits)
    mx = jnp.max(logits, axis=1, keepdims=True)
    ex = jnp.exp(logits - mx)
    nsc = ex / jnp.sum(ex, axis=1, keepdims=True)
    score_ref[...] = nsc
    supo_ref[...] = lax.dot_general(nsc, s3, (((1,), (1,)), ((0,), (0,))))
    sdst_ref[...] = jnp.dot(sent, wsn_ref[...],
                            preferred_element_type=jnp.float32)


def _super_layer(sent_new, super_h, Wsup, Ws_next):
    n = sent_new.shape[0]
    Bn = super_h.shape[0]
    grid = Bn // DOC_BLK
    return pl.pallas_call(
        _super_layer_body,
        grid=(grid,),
        in_specs=[
            pl.BlockSpec((DOC_BLK * SN, DD), lambda i: (i, 0)),
            pl.BlockSpec((DOC_BLK, DD), lambda i: (i, 0)),
            pl.BlockSpec((DD, DD), lambda i: (0, 0)),
            pl.BlockSpec((DD, DD), lambda i: (0, 0)),
        ],
        out_specs=[
            pl.BlockSpec((DOC_BLK, SN), lambda i: (i, 0)),
            pl.BlockSpec((DOC_BLK, DD), lambda i: (i, 0)),
            pl.BlockSpec((DOC_BLK * SN, DD), lambda i: (i, 0)),
        ],
        out_shape=[
            jax.ShapeDtypeStruct((Bn, SN), jnp.float32),
            jax.ShapeDtypeStruct((Bn, DD), jnp.float32),
            jax.ShapeDtypeStruct((n, DD), jnp.float32),
        ],
        interpret=_INTERPRET,
    )(sent_new, super_h, Wsup, Ws_next)


def _pair_cls_body(sent_ref, score_ref, wc1_ref, bc1_ref, wc2_ref, bc2_ref,
                   out_ref, pair_ref):
    Bn = score_ref.shape[0]

    def body1(b, ssq):
        sb = sent_ref[pl.ds(b * SN, SN), :]
        scb = score_ref[pl.ds(b, 1), :]
        G = lax.dot_general(sb, sb, (((1,), (1,)), ((), ())))
        wt = lax.dot_general(scb, scb, (((0,), (0,)), ((), ())))
        pairb = jax.nn.sigmoid(wt * G)
        pair_ref[pl.ds(b * SN, SN), :] = pairb
        return ssq + pairb * pairb

    ssq = lax.fori_loop(0, Bn, body1, jnp.zeros((SN, SN), jnp.float32))
    nrm = jnp.sqrt(ssq) + 1e-12

    def body2(b, carry):
        sb = sent_ref[pl.ds(b * SN, SN), :]
        pairb = pair_ref[pl.ds(b * SN, SN), :] / nrm
        m = jnp.dot(pairb, sb, preferred_element_type=jnp.float32)
        h = jnp.maximum(
            jnp.dot(m, wc1_ref[...], preferred_element_type=jnp.float32)
            + bc1_ref[...], 0.0)
        out_ref[pl.ds(b * SN, SN), :] = (
            jnp.dot(h, wc2_ref[...], preferred_element_type=jnp.float32)
            + bc2_ref[...])
        return carry

    lax.fori_loop(0, Bn, body2, 0)


def _pair_cls(sent_h, new_score, Wc1, bc1, Wc2, bc2):
    n = sent_h.shape[0]
    Bn = new_score.shape[0]
    return pl.pallas_call(
        _pair_cls_body,
        out_shape=jax.ShapeDtypeStruct((n, 2), jnp.float32),
        scratch_shapes=[pltpu.VMEM((n, SN), jnp.float32)],
        interpret=_INTERPRET,
    )(sent_h, new_score, Wc1, bc1.reshape(1, -1), Wc2, bc2.reshape(1, -1))


def _sc_mesh():
    return plsc.VectorSubcoreMesh(core_axis_name="c", subcore_axis_name="s",
                                  num_cores=NC, num_subcores=NS)


def _embed_stage(wids, sent_wids, emb):
    nw = wids.shape[0]
    ns = sent_wids.shape[0]
    LW = sent_wids.shape[1]
    wpw = nw // NW
    spw = ns // NW
    CH = 64
    nch = wpw // CH

    @functools.partial(
        pl.kernel,
        out_type=(jax.ShapeDtypeStruct((nw, DD), jnp.float32),
                  jax.ShapeDtypeStruct((ns, DD), jnp.float32)),
        mesh=_sc_mesh(),
        scratch_types=[
            pltpu.VMEM((CH,), jnp.int32),
            pltpu.VMEM((CH, DD), jnp.float32),
            pltpu.VMEM((spw, LW), jnp.int32),
            pltpu.VMEM((LW, DD), jnp.float32),
            pltpu.VMEM((LW, DD), jnp.float32),
            pltpu.VMEM((spw, DD), jnp.float32),
            pltpu.SemaphoreType.DMA,
            pltpu.SemaphoreType.DMA,
            pltpu.SemaphoreType.DMA,
        ],
        compiler_params=pltpu.CompilerParams(needs_layout_passes=False),
        interpret=_INTERPRET,
    )
    def k(wids_hbm, swids_hbm, emb_hbm, wh_hbm, sh_hbm,
          widx_v, wrows_v, swids_v, srows0_v, srows1_v, sout_v,
          sem, sems0, sems1):
        w = lax.axis_index("s") * NC + lax.axis_index("c")

        def wchunk(kk, carry):
            base = pl.multiple_of((w * nch + kk) * CH, 8)
            pltpu.sync_copy(wids_hbm.at[pl.ds(base, CH)], widx_v)
            pltpu.async_copy(emb_hbm.at[widx_v], wrows_v, sem).wait()
            pltpu.sync_copy(wrows_v, wh_hbm.at[pl.ds(base, CH)])
            return carry

        lax.fori_loop(0, nch, wchunk, 0)

        s0 = pl.multiple_of(w * spw, 8)
        pltpu.sync_copy(swids_hbm.at[pl.ds(s0, spw)], swids_v)

        def sgather(i, buf, sm):
            pltpu.async_copy(emb_hbm.at[swids_v.at[jnp.minimum(i, spw - 1)]],
                             buf, sm)

        def saccum(i, buf):
            for g in range(8):
                a = buf[0, pl.ds(g * 16, 16)]
                for j in range(1, LW):
                    a = a + buf[j, pl.ds(g * 16, 16)]
                sout_v[i, pl.ds(g * 16, 16)] = a * (1.0 / LW)

        def swait(buf, sm):
            pltpu.make_async_copy(emb_hbm.at[swids_v.at[0]], buf, sm).wait()

        sgather(0, srows0_v, sems0)

        def spair(p, carry):
            swait(srows0_v, sems0)
            sgather(2 * p + 1, srows1_v, sems1)
            saccum(2 * p, srows0_v)
            swait(srows1_v, sems1)
            sgather(2 * p + 2, srows0_v, sems0)
            saccum(2 * p + 1, srows1_v)
            return carry

        lax.fori_loop(0, spw // 2, spair, 0)
        swait(srows0_v, sems0)
        pltpu.sync_copy(sout_v, sh_hbm.at[pl.ds(s0, spw)])

    return k(wids, sent_wids, emb)


_GDN = lax.GatherDimensionNumbers(
    offset_dims=(), collapsed_slice_dims=(0,), start_index_map=(0,))


def _shuf(x, idx):
    return lax.gather(x, idx[:, None], _GDN, slice_sizes=(1,),
                      mode=lax.GatherScatterMode.PROMISE_IN_BOUNDS)


def _tree_max(v, iota):
    for sh in (1, 2, 4, 8):
        v = jnp.maximum(v, _shuf(v, lax.bitwise_xor(iota, sh)))
    return v


def _tree_sum(v, iota):
    for sh in (1, 2, 4, 8):
        v = v + _shuf(v, lax.bitwise_xor(iota, sh))
    return v


def _edge_stage(wsrc, sdst, edge_src, row_start):
    n_sent = sdst.shape[0]
    spw = n_sent // NW
    K = 1
    CH = 16 * K

    @functools.partial(
        pl.kernel,
        out_type=jax.ShapeDtypeStruct((n_sent, DD), jnp.float32),
        mesh=_sc_mesh(),
        scratch_types=[
            pltpu.VMEM((spw + 16,), jnp.int32),
            pltpu.VMEM((spw, DD), jnp.float32),
            pltpu.VMEM((spw, DD), jnp.float32),
            pltpu.VMEM((CH,), jnp.int32),
            pltpu.VMEM((CH,), jnp.int32),
            pltpu.VMEM((CH,), jnp.int32),
            pltpu.VMEM((CH,), jnp.int32),
            pltpu.VMEM((CH, DD), jnp.float32),
            pltpu.VMEM((CH, DD), jnp.float32),
            pltpu.VMEM((CH, DD), jnp.float32),
            pltpu.VMEM((CH, DD), jnp.float32),
            pltpu.VMEM((CH, 16), jnp.float32),
            pltpu.VMEM((8, 16), jnp.float32),
            pltpu.VMEM((8, 16), jnp.float32),
            pltpu.SemaphoreType.DMA,
            pltpu.SemaphoreType.DMA,
            pltpu.SemaphoreType.DMA,
            pltpu.SemaphoreType.DMA,
            pltpu.SemaphoreType.DMA,
            pltpu.SemaphoreType.DMA,
            pltpu.SemaphoreType.DMA,
            pltpu.SemaphoreType.DMA,
        ],
        compiler_params=pltpu.CompilerParams(needs_layout_passes=False),
        interpret=_INTERPRET,
    )
    def k(wsrc_hbm, sdst_hbm, esrc_hbm, rs_hbm, out_hbm,
          rs_s, sdst_v, out_v, iA0, iA1, iB0, iB1, rA0, rA1, rB0, rB1,
          tbuf_v, accA_v, accB_v, smA0, smA1, smB0, smB1,
          siA0, siA1, siB0, siB1):
        w = lax.axis_index("s") * NC + lax.axis_index("c")
        s0 = pl.multiple_of(w * spw, 8)
        pltpu.sync_copy(rs_hbm.at[pl.ds(s0, spw + 16)], rs_s)
        pltpu.sync_copy(sdst_hbm.at[pl.ds(s0, spw)], sdst_v)
        iota = lax.broadcasted_iota(jnp.int32, (16,), 0)
        ne = esrc_hbm.shape[0]
        bmax = (ne - CH) // CH * CH

        def dma_base(c, base0):
            return pl.multiple_of(jnp.minimum(base0 + c * CH, bmax), 8)

        def issue_idx(c, base0, idx_v, sem):
            pltpu.async_copy(esrc_hbm.at[pl.ds(dma_base(c, base0), CH)],
                             idx_v, sem)

        def issue_rows(idx_v, rows_v, sem):
            pltpu.async_copy(wsrc_hbm.at[idx_v], rows_v, sem)

        def wait_rows(rows_v, sem):
            pltpu.make_async_copy(wsrc_hbm.at[iA0], rows_v, sem).wait()

        def wait_idx(idx_v, sem):
            pltpu.make_async_copy(esrc_hbm.at[pl.ds(0, CH)], idx_v, sem).wait()

        def compute(c, rows_v, st, lo, hi, base0, sd, acc_v):
            m, l = st
            gid = base0 + c * CH + iota
            valid = (gid >= lo) & (gid < hi)
            for j in range(CH):
                t = rows_v[j, pl.ds(0, 16)] * sd[0]
                for g in range(1, 8):
                    t = t + rows_v[j, pl.ds(g * 16, 16)] * sd[g]
                tbuf_v[j, :] = t
            e16 = plsc.load_gather(tbuf_v, [iota, jnp.zeros((16,), jnp.int32)])
            for p in range(1, 16):
                e16 = e16 + plsc.load_gather(
                    tbuf_v, [iota, jnp.full((16,), p, jnp.int32)])
            e16 = e16 * (DD ** -0.5)
            e16 = jnp.where(e16 >= 0, e16, 0.01 * e16)
            e16 = jnp.where(valid, e16, -1e30)
            m_new = jnp.maximum(m, _tree_max(e16, iota))
            w16 = jnp.where(valid, jnp.exp(e16 - m_new), 0.0)
            scale = jnp.exp(m - m_new)
            l_new = l * scale + _tree_sum(w16, iota)
            contrib = [None] * 8
            for j in range(16):
                wj = _shuf(w16, jnp.full((16,), j, jnp.int32))
                for g in range(8):
                    term = rows_v[j, pl.ds(g * 16, 16)] * wj
                    contrib[g] = term if contrib[g] is None else contrib[g] + term
            for g in range(8):
                acc_v[g, :] = acc_v[g, :] * scale + contrib[g]
            return (m_new, l_new)

        def sent_pair_body(p2, carry):
            ia = 2 * p2
            ib = 2 * p2 + 1
            rsv = rs_s[pl.ds(ia, 16)]
            loA, hiA, hiB = rsv[0], rsv[1], rsv[2]
            loB = hiA
            bA = lax.bitwise_and(loA, -CH)
            bB = lax.bitwise_and(loB, -CH)
            nchkA = (hiA - bA + CH - 1) // CH
            nchkB = (hiB - bB + CH - 1) // CH
            Q = (jnp.maximum(nchkA, nchkB) + 1) // 2
            sdA = [sdst_v[ia, pl.ds(g * 16, 16)] for g in range(8)]
            sdB = [sdst_v[ib, pl.ds(g * 16, 16)] for g in range(8)]
            zero = jnp.zeros((16,), jnp.float32)
            for g in range(8):
                accA_v[g, :] = zero
                accB_v[g, :] = zero

            issue_idx(0, bA, iA0, siA0)
            issue_idx(0, bB, iB0, siB0)
            wait_idx(iA0, siA0)
            issue_rows(iA0, rA0, smA0)
            issue_idx(1, bA, iA1, siA1)
            wait_idx(iB0, siB0)
            issue_rows(iB0, rB0, smB0)
            issue_idx(1, bB, iB1, siB1)

            def qbody(q, st):
                stA = st[:2]
                stB = st[2:]
                wait_idx(iA1, siA1)
                wait_rows(rA0, smA0)
                issue_idx(2 * q + 2, bA, iA0, siA0)
                issue_rows(iA1, rA1, smA1)
                wait_idx(iB1, siB1)
                wait_rows(rB0, smB0)
                issue_idx(2 * q + 2, bB, iB0, siB0)
                issue_rows(iB1, rB1, smB1)
                stA = compute(2 * q, rA0, stA, loA, hiA, bA, sdA, accA_v)
                stB = compute(2 * q, rB0, stB, loB, hiB, bB, sdB, accB_v)
                wait_idx(iA0, siA0)
                wait_rows(rA1, smA1)
                issue_idx(2 * q + 3, bA, iA1, siA1)
                issue_rows(iA0, rA0, smA0)
                wait_idx(iB0, siB0)
                wait_rows(rB1, smB1)
                issue_idx(2 * q + 3, bB, iB1, siB1)
                issue_rows(iB0, rB0, smB0)
                stA = compute(2 * q + 1, rA1, stA, loA, hiA, bA, sdA, accA_v)
                stB = compute(2 * q + 1, rB1, stB, loB, hiB, bB, sdB, accB_v)
                return (*stA, *stB)

            init1 = (jnp.full((16,), -1e30, jnp.float32),
                     jnp.zeros((16,), jnp.float32))
            res = lax.fori_loop(0, Q, qbody, init1 + init1)
            wait_idx(iA1, siA1)
            wait_rows(rA0, smA0)
            wait_idx(iB1, siB1)
            wait_rows(rB0, smB0)
            invA = 1.0 / (res[1] + 1e-9)
            invB = 1.0 / (res[3] + 1e-9)
            for g in range(8):
                out_v[ia, pl.ds(g * 16, 16)] = jnp.maximum(
                    sdA[g] + accA_v[g, :] * invA, 0.0)
                out_v[ib, pl.ds(g * 16, 16)] = jnp.maximum(
                    sdB[g] + accB_v[g, :] * invB, 0.0)
            return carry

        lax.fori_loop(0, spw // 2, sent_pair_body, 0)
        pltpu.sync_copy(out_v, out_hbm.at[pl.ds(s0, spw)])

    return k(wsrc, sdst, edge_src, row_start)


def kernel(wids, sent_wids, edge_src, edge_dst, score, emb,
           Ww0, Ws0, Wsup0, Ww1, Ws1, Wsup1, Ww2, Ws2, Wsup2,
           Wc1, bc1, Wc2, bc2):
    n_sent = sent_wids.shape[0]

    row_start = jnp.searchsorted(
        edge_dst, jnp.arange(n_sent + 1, dtype=edge_dst.dtype)
    ).astype(jnp.int32)
    row_start = jnp.concatenate(
        [row_start, jnp.full((15,), edge_src.shape[0], jnp.int32)])

    word_h, sent_h = _embed_stage(wids, sent_wids, emb)
    wsrc0, wsrc1, wsrc2 = _wsrc_chain(word_h, Ww0, Ww1, Ww2)
    super_h, sdst = _init_super(sent_h, score, Ws0)

    wsrcs = (wsrc0, wsrc1, wsrc2)
    wsups = (Wsup0, Wsup1, Wsup2)
    ws_next = (Ws1, Ws2, Ws0)
    new_score = score
    for l in range(3):
        sent_new = _edge_stage(wsrcs[l], sdst, edge_src, row_start)
        new_score, super_h, sdst = _super_layer(
            sent_new, super_h, wsups[l], ws_next[l])
        sent_h = sent_new

    sent_pair_out = _pair_cls(sent_h, new_score, Wc1, bc1, Wc2, bc2)
    return (new_score, sent_h, super_h, sent_pair_out)


# speedup vs baseline: 1.1161x; 1.1161x over previous
import functools
import jax
import jax.numpy as jnp
from jax import lax
from jax.experimental import pallas as pl
from jax.experimental.pallas import tpu as pltpu
from jax.experimental.pallas import tpu_sc as plsc

DD = 128
SN = 40
DOC_BLK = 8
NC = 2
NS = 16
NW = NC * NS


def _wsrc_chain_body(x_ref, w0_ref, w1_ref, w2_ref, o0_ref, o1_ref, o2_ref):
    x = x_ref[...]
    a0 = jnp.dot(x, w0_ref[...], preferred_element_type=jnp.float32)
    o0_ref[...] = a0
    a1 = jnp.dot(jnp.maximum(a0, 0.0), w1_ref[...],
                 preferred_element_type=jnp.float32)
    o1_ref[...] = a1
    o2_ref[...] = jnp.dot(jnp.maximum(a1, 0.0), w2_ref[...],
                          preferred_element_type=jnp.float32)


def _wsrc_chain(word_h, Ww0, Ww1, Ww2):
    n = word_h.shape[0]
    blk = 1024
    grid = n // blk
    w_spec = pl.BlockSpec((DD, DD), lambda i: (0, 0))
    x_spec = pl.BlockSpec((blk, DD), lambda i: (i, 0))
    shp = jax.ShapeDtypeStruct((n, DD), jnp.float32)
    return pl.pallas_call(
        _wsrc_chain_body,
        grid=(grid,),
        in_specs=[x_spec, w_spec, w_spec, w_spec],
        out_specs=[x_spec, x_spec, x_spec],
        out_shape=[shp, shp, shp],
    )(word_h, Ww0, Ww1, Ww2)


def _init_super_body(sent_ref, score_ref, ws_ref, sup_ref, sdst_ref):
    sent = sent_ref[...]
    sc = score_ref[...]
    s3 = sent.reshape(DOC_BLK, SN, DD)
    sup_ref[...] = lax.dot_general(sc, s3, (((1,), (1,)), ((0,), (0,))))
    sdst_ref[...] = jnp.dot(sent, ws_ref[...],
                            preferred_element_type=jnp.float32)


def _init_super(sent_h, score, Ws0):
    n = sent_h.shape[0]
    Bn = score.shape[0]
    grid = Bn // DOC_BLK
    return pl.pallas_call(
        _init_super_body,
        grid=(grid,),
        in_specs=[
            pl.BlockSpec((DOC_BLK * SN, DD), lambda i: (i, 0)),
            pl.BlockSpec((DOC_BLK, SN), lambda i: (i, 0)),
            pl.BlockSpec((DD, DD), lambda i: (0, 0)),
        ],
        out_specs=[
            pl.BlockSpec((DOC_BLK, DD), lambda i: (i, 0)),
            pl.BlockSpec((DOC_BLK * SN, DD), lambda i: (i, 0)),
        ],
        out_shape=[
            jax.ShapeDtypeStruct((Bn, DD), jnp.float32),
            jax.ShapeDtypeStruct((n, DD), jnp.float32),
        ],
    )(sent_h, score, Ws0)


def _super_layer_body(sent_ref, sup_ref, wsup_ref, wsn_ref,
                      score_ref, supo_ref, sdst_ref):
    sent = sent_ref[...]
    sup = sup_ref[...]
    proj = jnp.dot(sup, wsup_ref[...], preferred_element_type=jnp.float32)
    s3 = sent.reshape(DOC_BLK, SN, DD)
    logits = lax.dot_general(s3, proj, (((2,), (1,)), ((0,), (0,))))
    logits = logits * (DD ** -0.5)
    logits = jnp.where(logits >= 0, logits, 0.01 * logits)
    mx = jnp.max(logits, axis=1, keepdims=True)
    ex = jnp.exp(logits - mx)
    nsc = ex / jnp.sum(ex, axis=1, keepdims=True)
    score_ref[...] = nsc
    supo_ref[...] = lax.dot_general(nsc, s3, (((1,), (1,)), ((0,), (0,))))
    sdst_ref[...] = jnp.dot(sent, wsn_ref[...],
                            preferred_element_type=jnp.float32)


def _super_layer(sent_new, super_h, Wsup, Ws_next):
    n = sent_new.shape[0]
    Bn = super_h.shape[0]
    grid = Bn // DOC_BLK
    return pl.pallas_call(
        _super_layer_body,
        grid=(grid,),
        in_specs=[
            pl.BlockSpec((DOC_BLK * SN, DD), lambda i: (i, 0)),
            pl.BlockSpec((DOC_BLK, DD), lambda i: (i, 0)),
            pl.BlockSpec((DD, DD), lambda i: (0, 0)),
            pl.BlockSpec((DD, DD), lambda i: (0, 0)),
        ],
        out_specs=[
            pl.BlockSpec((DOC_BLK, SN), lambda i: (i, 0)),
            pl.BlockSpec((DOC_BLK, DD), lambda i: (i, 0)),
            pl.BlockSpec((DOC_BLK * SN, DD), lambda i: (i, 0)),
        ],
        out_shape=[
            jax.ShapeDtypeStruct((Bn, SN), jnp.float32),
            jax.ShapeDtypeStruct((Bn, DD), jnp.float32),
            jax.ShapeDtypeStruct((n, DD), jnp.float32),
        ],
    )(sent_new, super_h, Wsup, Ws_next)


def _pair_cls_body(sent_ref, score_ref, wc1_ref, bc1_ref, wc2_ref, bc2_ref,
                   out_ref, pair_ref):
    Bn = score_ref.shape[0]

    def body1(b, ssq):
        sb = sent_ref[pl.ds(b * SN, SN), :]
        scb = score_ref[pl.ds(b, 1), :]
        G = lax.dot_general(sb, sb, (((1,), (1,)), ((), ())))
        wt = lax.dot_general(scb, scb, (((0,), (0,)), ((), ())))
        pairb = jax.nn.sigmoid(wt * G)
        pair_ref[pl.ds(b * SN, SN), :] = pairb
        return ssq + pairb * pairb

    ssq = lax.fori_loop(0, Bn, body1, jnp.zeros((SN, SN), jnp.float32))
    nrm = jnp.sqrt(ssq) + 1e-12

    def body2(b, carry):
        sb = sent_ref[pl.ds(b * SN, SN), :]
        pairb = pair_ref[pl.ds(b * SN, SN), :] / nrm
        m = jnp.dot(pairb, sb, preferred_element_type=jnp.float32)
        h = jnp.maximum(
            jnp.dot(m, wc1_ref[...], preferred_element_type=jnp.float32)
            + bc1_ref[...], 0.0)
        out_ref[pl.ds(b * SN, SN), :] = (
            jnp.dot(h, wc2_ref[...], preferred_element_type=jnp.float32)
            + bc2_ref[...])
        return carry

    lax.fori_loop(0, Bn, body2, 0)


def _pair_cls(sent_h, new_score, Wc1, bc1, Wc2, bc2):
    n = sent_h.shape[0]
    Bn = new_score.shape[0]
    return pl.pallas_call(
        _pair_cls_body,
        out_shape=jax.ShapeDtypeStruct((n, 2), jnp.float32),
        scratch_shapes=[pltpu.VMEM((n, SN), jnp.float32)],
    )(sent_h, new_score, Wc1, bc1.reshape(1, -1), Wc2, bc2.reshape(1, -1))


def _sc_mesh():
    return plsc.VectorSubcoreMesh(core_axis_name="c", subcore_axis_name="s",
                                  num_cores=NC, num_subcores=NS)


def _embed_stage(wids, sent_wids, emb):
    nw = wids.shape[0]
    ns = sent_wids.shape[0]
    LW = sent_wids.shape[1]
    wpw = nw // NW
    spw = ns // NW
    CH = 64
    nch = wpw // CH

    @functools.partial(
        pl.kernel,
        out_type=(jax.ShapeDtypeStruct((nw, DD), jnp.float32),
                  jax.ShapeDtypeStruct((ns, DD), jnp.float32)),
        mesh=_sc_mesh(),
        scratch_types=[
            pltpu.VMEM((CH,), jnp.int32),
            pltpu.VMEM((CH, DD), jnp.float32),
            pltpu.VMEM((spw, LW), jnp.int32),
            pltpu.VMEM((LW, DD), jnp.float32),
            pltpu.VMEM((LW, DD), jnp.float32),
            pltpu.VMEM((spw, DD), jnp.float32),
            pltpu.SemaphoreType.DMA,
            pltpu.SemaphoreType.DMA,
            pltpu.SemaphoreType.DMA,
        ],
        compiler_params=pltpu.CompilerParams(needs_layout_passes=False),
    )
    def k(wids_hbm, swids_hbm, emb_hbm, wh_hbm, sh_hbm,
          widx_v, wrows_v, swids_v, srows0_v, srows1_v, sout_v,
          sem, sems0, sems1):
        w = lax.axis_index("s") * NC + lax.axis_index("c")

        def wchunk(kk, carry):
            base = pl.multiple_of((w * nch + kk) * CH, 8)
            pltpu.sync_copy(wids_hbm.at[pl.ds(base, CH)], widx_v)
            pltpu.async_copy(emb_hbm.at[widx_v], wrows_v, sem).wait()
            pltpu.sync_copy(wrows_v, wh_hbm.at[pl.ds(base, CH)])
            return carry

        lax.fori_loop(0, nch, wchunk, 0)

        s0 = pl.multiple_of(w * spw, 8)
        pltpu.sync_copy(swids_hbm.at[pl.ds(s0, spw)], swids_v)

        def sgather(i, buf, sm):
            pltpu.async_copy(emb_hbm.at[swids_v.at[jnp.minimum(i, spw - 1)]],
                             buf, sm)

        def saccum(i, buf):
            for g in range(8):
                a = buf[0, pl.ds(g * 16, 16)]
                for j in range(1, LW):
                    a = a + buf[j, pl.ds(g * 16, 16)]
                sout_v[i, pl.ds(g * 16, 16)] = a * (1.0 / LW)

        def swait(buf, sm):
            pltpu.make_async_copy(emb_hbm.at[swids_v.at[0]], buf, sm).wait()

        sgather(0, srows0_v, sems0)

        def spair(p, carry):
            swait(srows0_v, sems0)
            sgather(2 * p + 1, srows1_v, sems1)
            saccum(2 * p, srows0_v)
            swait(srows1_v, sems1)
            sgather(2 * p + 2, srows0_v, sems0)
            saccum(2 * p + 1, srows1_v)
            return carry

        lax.fori_loop(0, spw // 2, spair, 0)
        swait(srows0_v, sems0)
        pltpu.sync_copy(sout_v, sh_hbm.at[pl.ds(s0, spw)])

    return k(wids, sent_wids, emb)


_GDN = lax.GatherDimensionNumbers(
    offset_dims=(), collapsed_slice_dims=(0,), start_index_map=(0,))


def _shuf(x, idx):
    return lax.gather(x, idx[:, None], _GDN, slice_sizes=(1,),
                      mode=lax.GatherScatterMode.PROMISE_IN_BOUNDS)


def _tree_max(v, iota):
    for sh in (1, 2, 4, 8):
        v = jnp.maximum(v, _shuf(v, lax.bitwise_xor(iota, sh)))
    return v


def _tree_sum(v, iota):
    for sh in (1, 2, 4, 8):
        v = v + _shuf(v, lax.bitwise_xor(iota, sh))
    return v


def _edge_stage(wsrc, sdst, edge_src, row_start):
    n_sent = sdst.shape[0]
    spw = n_sent // NW
    K = 1
    CH = 16 * K

    @functools.partial(
        pl.kernel,
        out_type=jax.ShapeDtypeStruct((n_sent, DD), jnp.float32),
        mesh=_sc_mesh(),
        scratch_types=[
            pltpu.VMEM((spw + 16,), jnp.int32),
            pltpu.VMEM((spw, DD), jnp.float32),
            pltpu.VMEM((spw, DD), jnp.float32),
            pltpu.VMEM((CH,), jnp.int32),
            pltpu.VMEM((CH,), jnp.int32),
            pltpu.VMEM((CH,), jnp.int32),
            pltpu.VMEM((CH,), jnp.int32),
            pltpu.VMEM((CH, DD), jnp.float32),
            pltpu.VMEM((CH, DD), jnp.float32),
            pltpu.VMEM((CH, DD), jnp.float32),
            pltpu.VMEM((CH, DD), jnp.float32),
            pltpu.VMEM((CH, 16), jnp.float32),
            pltpu.SemaphoreType.DMA,
            pltpu.SemaphoreType.DMA,
            pltpu.SemaphoreType.DMA,
            pltpu.SemaphoreType.DMA,
            pltpu.SemaphoreType.DMA,
            pltpu.SemaphoreType.DMA,
            pltpu.SemaphoreType.DMA,
            pltpu.SemaphoreType.DMA,
        ],
        compiler_params=pltpu.CompilerParams(needs_layout_passes=False),
    )
    def k(wsrc_hbm, sdst_hbm, esrc_hbm, rs_hbm, out_hbm,
          rs_s, sdst_v, out_v, iA0, iA1, iB0, iB1, rA0, rA1, rB0, rB1,
          tbuf_v, smA0, smA1, smB0, smB1, siA0, siA1, siB0, siB1):
        w = lax.axis_index("s") * NC + lax.axis_index("c")
        s0 = pl.multiple_of(w * spw, 8)
        pltpu.sync_copy(rs_hbm.at[pl.ds(s0, spw + 16)], rs_s)
        pltpu.sync_copy(sdst_hbm.at[pl.ds(s0, spw)], sdst_v)
        iota = lax.broadcasted_iota(jnp.int32, (16,), 0)
        ne = esrc_hbm.shape[0]
        bmax = (ne - CH) // CH * CH

        def dma_base(c, base0):
            return pl.multiple_of(jnp.minimum(base0 + c * CH, bmax), 8)

        def issue_idx(c, base0, idx_v, sem):
            pltpu.async_copy(esrc_hbm.at[pl.ds(dma_base(c, base0), CH)],
                             idx_v, sem)

        def issue_rows(idx_v, rows_v, sem):
            pltpu.async_copy(wsrc_hbm.at[idx_v], rows_v, sem)

        def wait_rows(rows_v, sem):
            pltpu.make_async_copy(wsrc_hbm.at[iA0], rows_v, sem).wait()

        def wait_idx(idx_v, sem):
            pltpu.make_async_copy(esrc_hbm.at[pl.ds(0, CH)], idx_v, sem).wait()

        def compute(c, rows_v, st, lo, hi, base0, sd):
            m, l = st[0], st[1]
            acc = list(st[2:])
            gid = base0 + c * CH + iota
            valid = (gid >= lo) & (gid < hi)
            for j in range(CH):
                t = rows_v[j, pl.ds(0, 16)] * sd[0]
                for g in range(1, 8):
                    t = t + rows_v[j, pl.ds(g * 16, 16)] * sd[g]
                tbuf_v[j, :] = t
            e16 = plsc.load_gather(tbuf_v, [iota, jnp.zeros((16,), jnp.int32)])
            for p in range(1, 16):
                e16 = e16 + plsc.load_gather(
                    tbuf_v, [iota, jnp.full((16,), p, jnp.int32)])
            e16 = e16 * (DD ** -0.5)
            e16 = jnp.where(e16 >= 0, e16, 0.01 * e16)
            e16 = jnp.where(valid, e16, -1e30)
            m_new = jnp.maximum(m, _tree_max(e16, iota))
            w16 = jnp.where(valid, jnp.exp(e16 - m_new), 0.0)
            scale = jnp.exp(m - m_new)
            l_new = l * scale + _tree_sum(w16, iota)
            new_acc = [acc[g] * scale for g in range(8)]
            for j in range(16):
                wj = _shuf(w16, jnp.full((16,), j, jnp.int32))
                for g in range(8):
                    new_acc[g] = new_acc[g] + rows_v[j, pl.ds(g * 16, 16)] * wj
            return (m_new, l_new, *new_acc)

        def sent_pair_body(p2, carry):
            ia = 2 * p2
            ib = 2 * p2 + 1
            rsv = rs_s[pl.ds(ia, 16)]
            loA, hiA, hiB = rsv[0], rsv[1], rsv[2]
            loB = hiA
            bA = lax.bitwise_and(loA, -CH)
            bB = lax.bitwise_and(loB, -CH)
            nchkA = (hiA - bA + CH - 1) // CH
            nchkB = (hiB - bB + CH - 1) // CH
            Q = (jnp.maximum(nchkA, nchkB) + 1) // 2
            sdA = [sdst_v[ia, pl.ds(g * 16, 16)] for g in range(8)]
            sdB = [sdst_v[ib, pl.ds(g * 16, 16)] for g in range(8)]
            issue_idx(0, bA, iA0, siA0)
            issue_idx(0, bB, iB0, siB0)
            wait_idx(iA0, siA0)
            issue_rows(iA0, rA0, smA0)
            issue_idx(1, bA, iA1, siA1)
            wait_idx(iB0, siB0)
            issue_rows(iB0, rB0, smB0)
            issue_idx(1, bB, iB1, siB1)

            def qbody(q, st):
                stA = st[:10]
                stB = st[10:]
                wait_idx(iA1, siA1)
                wait_rows(rA0, smA0)
                issue_idx(2 * q + 2, bA, iA0, siA0)
                issue_rows(iA1, rA1, smA1)
                wait_idx(iB1, siB1)
                wait_rows(rB0, smB0)
                issue_idx(2 * q + 2, bB, iB0, siB0)
                issue_rows(iB1, rB1, smB1)
                stA = compute(2 * q, rA0, stA, loA, hiA, bA, sdA)
                stB = compute(2 * q, rB0, stB, loB, hiB, bB, sdB)
                wait_idx(iA0, siA0)
                wait_rows(rA1, smA1)
                issue_idx(2 * q + 3, bA, iA1, siA1)
                issue_rows(iA0, rA0, smA0)
                wait_idx(iB0, siB0)
                wait_rows(rB1, smB1)
                issue_idx(2 * q + 3, bB, iB1, siB1)
                issue_rows(iB0, rB0, smB0)
                stA = compute(2 * q + 1, rA1, stA, loA, hiA, bA, sdA)
                stB = compute(2 * q + 1, rB1, stB, loB, hiB, bB, sdB)
                return (*stA, *stB)

            init1 = ((jnp.full((16,), -1e30, jnp.float32),
                      jnp.zeros((16,), jnp.float32))
                     + tuple(jnp.zeros((16,), jnp.float32) for _ in range(8)))
            res = lax.fori_loop(0, Q, qbody, init1 + init1)
            wait_idx(iA1, siA1)
            wait_rows(rA0, smA0)
            wait_idx(iB1, siB1)
            wait_rows(rB0, smB0)
            invA = 1.0 / (res[1] + 1e-9)
            invB = 1.0 / (res[11] + 1e-9)
            for g in range(8):
                out_v[ia, pl.ds(g * 16, 16)] = jnp.maximum(
                    sdA[g] + res[2 + g] * invA, 0.0)
                out_v[ib, pl.ds(g * 16, 16)] = jnp.maximum(
                    sdB[g] + res[12 + g] * invB, 0.0)
            return carry

        lax.fori_loop(0, spw // 2, sent_pair_body, 0)
        pltpu.sync_copy(out_v, out_hbm.at[pl.ds(s0, spw)])

    return k(wsrc, sdst, edge_src, row_start)


def kernel(wids, sent_wids, edge_src, edge_dst, score, emb,
           Ww0, Ws0, Wsup0, Ww1, Ws1, Wsup1, Ww2, Ws2, Wsup2,
           Wc1, bc1, Wc2, bc2):
    n_sent = sent_wids.shape[0]

    row_start = jnp.searchsorted(
        edge_dst, jnp.arange(n_sent + 1, dtype=edge_dst.dtype)
    ).astype(jnp.int32)
    row_start = jnp.concatenate(
        [row_start, jnp.full((15,), edge_src.shape[0], jnp.int32)])

    word_h, sent_h = _embed_stage(wids, sent_wids, emb)
    wsrc0, wsrc1, wsrc2 = _wsrc_chain(word_h, Ww0, Ww1, Ww2)
    super_h, sdst = _init_super(sent_h, score, Ws0)

    wsrcs = (wsrc0, wsrc1, wsrc2)
    wsups = (Wsup0, Wsup1, Wsup2)
    ws_next = (Ws1, Ws2, Ws0)
    new_score = score
    for l in range(3):
        sent_new = _edge_stage(wsrcs[l], sdst, edge_src, row_start)
        new_score, super_h, sdst = _super_layer(
            sent_new, super_h, wsups[l], ws_next[l])
        sent_h = sent_new

    sent_pair_out = _pair_cls(sent_h, new_score, Wc1, bc1, Wc2, bc2)
    return (new_score, sent_h, super_h, sent_pair_out)


# speedup vs baseline: 1.2024x; 1.0773x over previous
import functools
import jax
import jax.numpy as jnp
from jax import lax
from jax.experimental import pallas as pl
from jax.experimental.pallas import tpu as pltpu
from jax.experimental.pallas import tpu_sc as plsc

DD = 128
SN = 40
DOC_BLK = 8
NC = 2
NS = 16
NW = NC * NS


def _wsrc_chain_body(x_ref, w0_ref, w1_ref, w2_ref, o0_ref, o1_ref, o2_ref):
    x = x_ref[...]
    a0 = jnp.dot(x, w0_ref[...], preferred_element_type=jnp.float32)
    o0_ref[...] = a0
    a1 = jnp.dot(jnp.maximum(a0, 0.0), w1_ref[...],
                 preferred_element_type=jnp.float32)
    o1_ref[...] = a1
    o2_ref[...] = jnp.dot(jnp.maximum(a1, 0.0), w2_ref[...],
                          preferred_element_type=jnp.float32)


def _wsrc_chain(word_h, Ww0, Ww1, Ww2):
    n = word_h.shape[0]
    blk = 1024
    grid = n // blk
    w_spec = pl.BlockSpec((DD, DD), lambda i: (0, 0))
    x_spec = pl.BlockSpec((blk, DD), lambda i: (i, 0))
    shp = jax.ShapeDtypeStruct((n, DD), jnp.float32)
    return pl.pallas_call(
        _wsrc_chain_body,
        grid=(grid,),
        in_specs=[x_spec, w_spec, w_spec, w_spec],
        out_specs=[x_spec, x_spec, x_spec],
        out_shape=[shp, shp, shp],
    )(word_h, Ww0, Ww1, Ww2)


def _init_super_body(sent_ref, score_ref, ws_ref, sup_ref, sdst_ref):
    sent = sent_ref[...]
    sc = score_ref[...]
    s3 = sent.reshape(DOC_BLK, SN, DD)
    sup_ref[...] = lax.dot_general(sc, s3, (((1,), (1,)), ((0,), (0,))))
    sdst_ref[...] = jnp.dot(sent, ws_ref[...],
                            preferred_element_type=jnp.float32)


def _init_super(sent_h, score, Ws0):
    n = sent_h.shape[0]
    Bn = score.shape[0]
    grid = Bn // DOC_BLK
    return pl.pallas_call(
        _init_super_body,
        grid=(grid,),
        in_specs=[
            pl.BlockSpec((DOC_BLK * SN, DD), lambda i: (i, 0)),
            pl.BlockSpec((DOC_BLK, SN), lambda i: (i, 0)),
            pl.BlockSpec((DD, DD), lambda i: (0, 0)),
        ],
        out_specs=[
            pl.BlockSpec((DOC_BLK, DD), lambda i: (i, 0)),
            pl.BlockSpec((DOC_BLK * SN, DD), lambda i: (i, 0)),
        ],
        out_shape=[
            jax.ShapeDtypeStruct((Bn, DD), jnp.float32),
            jax.ShapeDtypeStruct((n, DD), jnp.float32),
        ],
    )(sent_h, score, Ws0)


def _super_layer_body(sent_ref, sup_ref, wsup_ref, wsn_ref,
                      score_ref, supo_ref, sdst_ref):
    sent = sent_ref[...]
    sup = sup_ref[...]
    proj = jnp.dot(sup, wsup_ref[...], preferred_element_type=jnp.float32)
    s3 = sent.reshape(DOC_BLK, SN, DD)
    logits = lax.dot_general(s3, proj, (((2,), (1,)), ((0,), (0,))))
    logits = logits * (DD ** -0.5)
    logits = jnp.where(logits >= 0, logits, 0.01 * logits)
    mx = jnp.max(logits, axis=1, keepdims=True)
    ex = jnp.exp(logits - mx)
    nsc = ex / jnp.sum(ex, axis=1, keepdims=True)
    score_ref[...] = nsc
    supo_ref[...] = lax.dot_general(nsc, s3, (((1,), (1,)), ((0,), (0,))))
    sdst_ref[...] = jnp.dot(sent, wsn_ref[...],
                            preferred_element_type=jnp.float32)


def _super_layer(sent_new, super_h, Wsup, Ws_next):
    n = sent_new.shape[0]
    Bn = super_h.shape[0]
    grid = Bn // DOC_BLK
    return pl.pallas_call(
        _super_layer_body,
        grid=(grid,),
        in_specs=[
            pl.BlockSpec((DOC_BLK * SN, DD), lambda i: (i, 0)),
            pl.BlockSpec((DOC_BLK, DD), lambda i: (i, 0)),
            pl.BlockSpec((DD, DD), lambda i: (0, 0)),
            pl.BlockSpec((DD, DD), lambda i: (0, 0)),
        ],
        out_specs=[
            pl.BlockSpec((DOC_BLK, SN), lambda i: (i, 0)),
            pl.BlockSpec((DOC_BLK, DD), lambda i: (i, 0)),
            pl.BlockSpec((DOC_BLK * SN, DD), lambda i: (i, 0)),
        ],
        out_shape=[
            jax.ShapeDtypeStruct((Bn, SN), jnp.float32),
            jax.ShapeDtypeStruct((Bn, DD), jnp.float32),
            jax.ShapeDtypeStruct((n, DD), jnp.float32),
        ],
    )(sent_new, super_h, Wsup, Ws_next)


def _pair_cls_body(sent_ref, score_ref, wc1_ref, bc1_ref, wc2_ref, bc2_ref,
                   out_ref, pair_ref):
    Bn = score_ref.shape[0]

    def body1(b, ssq):
        sb = sent_ref[pl.ds(b * SN, SN), :]
        scb = score_ref[pl.ds(b, 1), :]
        G = lax.dot_general(sb, sb, (((1,), (1,)), ((), ())))
        wt = lax.dot_general(scb, scb, (((0,), (0,)), ((), ())))
        pairb = jax.nn.sigmoid(wt * G)
        pair_ref[pl.ds(b * SN, SN), :] = pairb
        return ssq + pairb * pairb

    ssq = lax.fori_loop(0, Bn, body1, jnp.zeros((SN, SN), jnp.float32))
    nrm = jnp.sqrt(ssq) + 1e-12

    def body2(b, carry):
        sb = sent_ref[pl.ds(b * SN, SN), :]
        pairb = pair_ref[pl.ds(b * SN, SN), :] / nrm
        m = jnp.dot(pairb, sb, preferred_element_type=jnp.float32)
        h = jnp.maximum(
            jnp.dot(m, wc1_ref[...], preferred_element_type=jnp.float32)
            + bc1_ref[...], 0.0)
        out_ref[pl.ds(b * SN, SN), :] = (
            jnp.dot(h, wc2_ref[...], preferred_element_type=jnp.float32)
            + bc2_ref[...])
        return carry

    lax.fori_loop(0, Bn, body2, 0)


def _pair_cls(sent_h, new_score, Wc1, bc1, Wc2, bc2):
    n = sent_h.shape[0]
    Bn = new_score.shape[0]
    return pl.pallas_call(
        _pair_cls_body,
        out_shape=jax.ShapeDtypeStruct((n, 2), jnp.float32),
        scratch_shapes=[pltpu.VMEM((n, SN), jnp.float32)],
    )(sent_h, new_score, Wc1, bc1.reshape(1, -1), Wc2, bc2.reshape(1, -1))


def _sc_mesh():
    return plsc.VectorSubcoreMesh(core_axis_name="c", subcore_axis_name="s",
                                  num_cores=NC, num_subcores=NS)


def _embed_stage(wids, sent_wids, emb):
    nw = wids.shape[0]
    ns = sent_wids.shape[0]
    LW = sent_wids.shape[1]
    wpw = nw // NW
    spw = ns // NW
    CH = 64
    nch = wpw // CH

    @functools.partial(
        pl.kernel,
        out_type=(jax.ShapeDtypeStruct((nw, DD), jnp.float32),
                  jax.ShapeDtypeStruct((ns, DD), jnp.float32)),
        mesh=_sc_mesh(),
        scratch_types=[
            pltpu.VMEM((CH,), jnp.int32),
            pltpu.VMEM((CH, DD), jnp.float32),
            pltpu.VMEM((spw, LW), jnp.int32),
            pltpu.VMEM((LW, DD), jnp.float32),
            pltpu.VMEM((LW, DD), jnp.float32),
            pltpu.VMEM((spw, DD), jnp.float32),
            pltpu.SemaphoreType.DMA,
            pltpu.SemaphoreType.DMA,
            pltpu.SemaphoreType.DMA,
        ],
        compiler_params=pltpu.CompilerParams(needs_layout_passes=False),
    )
    def k(wids_hbm, swids_hbm, emb_hbm, wh_hbm, sh_hbm,
          widx_v, wrows_v, swids_v, srows0_v, srows1_v, sout_v,
          sem, sems0, sems1):
        w = lax.axis_index("s") * NC + lax.axis_index("c")

        def wchunk(kk, carry):
            base = pl.multiple_of((w * nch + kk) * CH, 8)
            pltpu.sync_copy(wids_hbm.at[pl.ds(base, CH)], widx_v)
            pltpu.async_copy(emb_hbm.at[widx_v], wrows_v, sem).wait()
            pltpu.sync_copy(wrows_v, wh_hbm.at[pl.ds(base, CH)])
            return carry

        lax.fori_loop(0, nch, wchunk, 0)

        s0 = pl.multiple_of(w * spw, 8)
        pltpu.sync_copy(swids_hbm.at[pl.ds(s0, spw)], swids_v)

        def sgather(i, buf, sm):
            pltpu.async_copy(emb_hbm.at[swids_v.at[jnp.minimum(i, spw - 1)]],
                             buf, sm)

        def saccum(i, buf):
            for g in range(8):
                a = buf[0, pl.ds(g * 16, 16)]
                for j in range(1, LW):
                    a = a + buf[j, pl.ds(g * 16, 16)]
                sout_v[i, pl.ds(g * 16, 16)] = a * (1.0 / LW)

        def swait(buf, sm):
            pltpu.make_async_copy(emb_hbm.at[swids_v.at[0]], buf, sm).wait()

        sgather(0, srows0_v, sems0)

        def spair(p, carry):
            swait(srows0_v, sems0)
            sgather(2 * p + 1, srows1_v, sems1)
            saccum(2 * p, srows0_v)
            swait(srows1_v, sems1)
            sgather(2 * p + 2, srows0_v, sems0)
            saccum(2 * p + 1, srows1_v)
            return carry

        lax.fori_loop(0, spw // 2, spair, 0)
        swait(srows0_v, sems0)
        pltpu.sync_copy(sout_v, sh_hbm.at[pl.ds(s0, spw)])

    return k(wids, sent_wids, emb)


_GDN = lax.GatherDimensionNumbers(
    offset_dims=(), collapsed_slice_dims=(0,), start_index_map=(0,))


def _shuf(x, idx):
    return lax.gather(x, idx[:, None], _GDN, slice_sizes=(1,),
                      mode=lax.GatherScatterMode.PROMISE_IN_BOUNDS)


def _tree_max(v, iota):
    for sh in (1, 2, 4, 8):
        v = jnp.maximum(v, _shuf(v, lax.bitwise_xor(iota, sh)))
    return v


def _tree_sum(v, iota):
    for sh in (1, 2, 4, 8):
        v = v + _shuf(v, lax.bitwise_xor(iota, sh))
    return v


def _edge_stage(wsrc, sdst, edge_src, row_start):
    n_sent = sdst.shape[0]
    spw = n_sent // NW
    K = 1
    CH = 16 * K

    @functools.partial(
        pl.kernel,
        out_type=jax.ShapeDtypeStruct((n_sent, DD), jnp.float32),
        mesh=_sc_mesh(),
        scratch_types=[
            pltpu.VMEM((spw + 16,), jnp.int32),
            pltpu.VMEM((spw, DD), jnp.float32),
            pltpu.VMEM((spw, DD), jnp.float32),
            pltpu.VMEM((CH,), jnp.int32),
            pltpu.VMEM((CH,), jnp.int32),
            pltpu.VMEM((CH,), jnp.int32),
            pltpu.VMEM((CH,), jnp.int32),
            pltpu.VMEM((CH, DD), jnp.float32),
            pltpu.VMEM((CH, DD), jnp.float32),
            pltpu.VMEM((CH, DD), jnp.float32),
            pltpu.VMEM((CH, DD), jnp.float32),
            pltpu.VMEM((CH, 16), jnp.float32),
            pltpu.SemaphoreType.DMA,
            pltpu.SemaphoreType.DMA,
            pltpu.SemaphoreType.DMA,
            pltpu.SemaphoreType.DMA,
            pltpu.SemaphoreType.DMA,
            pltpu.SemaphoreType.DMA,
            pltpu.SemaphoreType.DMA,
            pltpu.SemaphoreType.DMA,
        ],
        compiler_params=pltpu.CompilerParams(needs_layout_passes=False),
    )
    def k(wsrc_hbm, sdst_hbm, esrc_hbm, rs_hbm, out_hbm,
          rs_s, sdst_v, out_v, iA0, iA1, iB0, iB1, rA0, rA1, rB0, rB1,
          tbuf_v, smA0, smA1, smB0, smB1, siA0, siA1, siB0, siB1):
        w = lax.axis_index("s") * NC + lax.axis_index("c")
        s0 = pl.multiple_of(w * spw, 8)
        pltpu.sync_copy(rs_hbm.at[pl.ds(s0, spw + 16)], rs_s)
        pltpu.sync_copy(sdst_hbm.at[pl.ds(s0, spw)], sdst_v)
        iota = lax.broadcasted_iota(jnp.int32, (16,), 0)
        ne = esrc_hbm.shape[0]
        bmax = (ne - CH) // CH * CH

        def dma_base(c, base0):
            return pl.multiple_of(jnp.minimum(base0 + c * CH, bmax), 8)

        def issue_idx(c, base0, idx_v, sem):
            pltpu.async_copy(esrc_hbm.at[pl.ds(dma_base(c, base0), CH)],
                             idx_v, sem)

        def issue_rows(idx_v, rows_v, sem):
            pltpu.async_copy(wsrc_hbm.at[idx_v], rows_v, sem)

        def wait_rows(rows_v, sem):
            pltpu.make_async_copy(wsrc_hbm.at[iA0], rows_v, sem).wait()

        def wait_idx(idx_v, sem):
            pltpu.make_async_copy(esrc_hbm.at[pl.ds(0, CH)], idx_v, sem).wait()

        def compute(c, rows_v, st, lo, hi, base0, sd):
            m, l = st[0], st[1]
            acc = list(st[2:])
            gid = base0 + c * CH + iota
            valid = (gid >= lo) & (gid < hi)
            for j in range(CH):
                t = rows_v[j, pl.ds(0, 16)] * sd[0]
                for g in range(1, 8):
                    t = t + rows_v[j, pl.ds(g * 16, 16)] * sd[g]
                tbuf_v[j, :] = t
            e16 = plsc.load_gather(tbuf_v, [iota, jnp.zeros((16,), jnp.int32)])
            for p in range(1, 16):
                e16 = e16 + plsc.load_gather(
                    tbuf_v, [iota, jnp.full((16,), p, jnp.int32)])
            e16 = e16 * (DD ** -0.5)
            e16 = jnp.where(e16 >= 0, e16, 0.01 * e16)
            e16 = jnp.where(valid, e16, -1e30)
            m_new = jnp.maximum(m, _tree_max(e16, iota))
            w16 = jnp.where(valid, jnp.exp(e16 - m_new), 0.0)
            scale = jnp.exp(m - m_new)
            l_new = l * scale + _tree_sum(w16, iota)
            new_acc = [acc[g] * scale for g in range(8)]
            for j in range(16):
                wj = _shuf(w16, jnp.full((16,), j, jnp.int32))
                for g in range(8):
                    new_acc[g] = new_acc[g] + rows_v[j, pl.ds(g * 16, 16)] * wj
            return (m_new, l_new, *new_acc)

        def sent_pair_body(p2, carry):
            ia = 2 * p2
            ib = 2 * p2 + 1
            rsv = rs_s[pl.ds(ia, 16)]
            loA, hiA, hiB = rsv[0], rsv[1], rsv[2]
            loB = hiA
            bA = lax.bitwise_and(loA, -8)
            bB = lax.bitwise_and(loB, -8)
            nchkA = (hiA - bA + CH - 1) // CH
            nchkB = (hiB - bB + CH - 1) // CH
            Q = (jnp.maximum(nchkA, nchkB) + 1) // 2
            sdA = [sdst_v[ia, pl.ds(g * 16, 16)] for g in range(8)]
            sdB = [sdst_v[ib, pl.ds(g * 16, 16)] for g in range(8)]
            issue_idx(0, bA, iA0, siA0)
            issue_idx(0, bB, iB0, siB0)
            wait_idx(iA0, siA0)
            issue_rows(iA0, rA0, smA0)
            issue_idx(1, bA, iA1, siA1)
            wait_idx(iB0, siB0)
            issue_rows(iB0, rB0, smB0)
            issue_idx(1, bB, iB1, siB1)

            def qbody(q, st):
                stA = st[:10]
                stB = st[10:]
                wait_idx(iA1, siA1)
                wait_rows(rA0, smA0)
                issue_idx(2 * q + 2, bA, iA0, siA0)
                issue_rows(iA1, rA1, smA1)
                wait_idx(iB1, siB1)
                wait_rows(rB0, smB0)
                issue_idx(2 * q + 2, bB, iB0, siB0)
                issue_rows(iB1, rB1, smB1)
                stA = compute(2 * q, rA0, stA, loA, hiA, bA, sdA)
                stB = compute(2 * q, rB0, stB, loB, hiB, bB, sdB)
                wait_idx(iA0, siA0)
                wait_rows(rA1, smA1)
                issue_idx(2 * q + 3, bA, iA1, siA1)
                issue_rows(iA0, rA0, smA0)
                wait_idx(iB0, siB0)
                wait_rows(rB1, smB1)
                issue_idx(2 * q + 3, bB, iB1, siB1)
                issue_rows(iB0, rB0, smB0)
                stA = compute(2 * q + 1, rA1, stA, loA, hiA, bA, sdA)
                stB = compute(2 * q + 1, rB1, stB, loB, hiB, bB, sdB)
                return (*stA, *stB)

            init1 = ((jnp.full((16,), -1e30, jnp.float32),
                      jnp.zeros((16,), jnp.float32))
                     + tuple(jnp.zeros((16,), jnp.float32) for _ in range(8)))
            res = lax.fori_loop(0, Q, qbody, init1 + init1)
            wait_idx(iA1, siA1)
            wait_rows(rA0, smA0)
            wait_idx(iB1, siB1)
            wait_rows(rB0, smB0)
            invA = 1.0 / (res[1] + 1e-9)
            invB = 1.0 / (res[11] + 1e-9)
            for g in range(8):
                out_v[ia, pl.ds(g * 16, 16)] = jnp.maximum(
                    sdA[g] + res[2 + g] * invA, 0.0)
                out_v[ib, pl.ds(g * 16, 16)] = jnp.maximum(
                    sdB[g] + res[12 + g] * invB, 0.0)
            return carry

        lax.fori_loop(0, spw // 2, sent_pair_body, 0)
        pltpu.sync_copy(out_v, out_hbm.at[pl.ds(s0, spw)])

    return k(wsrc, sdst, edge_src, row_start)


def kernel(wids, sent_wids, edge_src, edge_dst, score, emb,
           Ww0, Ws0, Wsup0, Ww1, Ws1, Wsup1, Ww2, Ws2, Wsup2,
           Wc1, bc1, Wc2, bc2):
    n_sent = sent_wids.shape[0]

    row_start = jnp.searchsorted(
        edge_dst, jnp.arange(n_sent + 1, dtype=edge_dst.dtype)
    ).astype(jnp.int32)
    row_start = jnp.concatenate(
        [row_start, jnp.full((15,), edge_src.shape[0], jnp.int32)])

    word_h, sent_h = _embed_stage(wids, sent_wids, emb)
    wsrc0, wsrc1, wsrc2 = _wsrc_chain(word_h, Ww0, Ww1, Ww2)
    super_h, sdst = _init_super(sent_h, score, Ws0)

    wsrcs = (wsrc0, wsrc1, wsrc2)
    wsups = (Wsup0, Wsup1, Wsup2)
    ws_next = (Ws1, Ws2, Ws0)
    new_score = score
    for l in range(3):
        sent_new = _edge_stage(wsrcs[l], sdst, edge_src, row_start)
        new_score, super_h, sdst = _super_layer(
            sent_new, super_h, wsups[l], ws_next[l])
        sent_h = sent_new

    sent_pair_out = _pair_cls(sent_h, new_score, Wc1, bc1, Wc2, bc2)
    return (new_score, sent_h, super_h, sent_pair_out)
